# Initial kernel scaffold; baseline (speedup 1.0000x reference)
import functools

import jax
import jax.numpy as jnp
from jax import lax
from jax.experimental import pallas as pl
from jax.experimental.pallas import tpu as pltpu
from jax.experimental.pallas import tpu_sc as plsc

_B = 16384
_C = 1000
_EPS = 0.1
_BCOEF = _EPS / (_C - 1)
_ACOEF = (1.0 - _EPS) - _BCOEF

_NC = 2
_NS = 16
_NW = _NC * _NS
_RPW = _B // _NW
_CHUNKS = _RPW // 128


@functools.partial(
    pl.kernel,
    mesh=plsc.VectorSubcoreMesh(core_axis_name="c", subcore_axis_name="s"),
    out_type=jax.ShapeDtypeStruct((_NW, 16), jnp.float32),
    scratch_types=[
        pltpu.VMEM((_RPW,), jnp.int32),
        pltpu.VMEM((_CHUNKS, 128), jnp.int32),
        pltpu.VMEM((_CHUNKS, 128), jnp.float32),
        pltpu.VMEM((16,), jnp.float32),
        pltpu.SemaphoreType.DMA,
    ],
)
def _sc_gather(flat_hbm, tgt_hbm, out_hbm, tgt_v, idx_v, g_v, acc_v, sem):
    wid = lax.axis_index("s") * _NC + lax.axis_index("c")
    base = wid * _RPW
    pltpu.sync_copy(tgt_hbm.at[pl.ds(base, _RPW)], tgt_v)
    for j in range(_CHUNKS):
        for k in range(8):
            i = j * 8 + k
            t = tgt_v[pl.ds(i * 16, 16)]
            rows = base + i * 16 + lax.iota(jnp.int32, 16)
            idx_v[j, pl.ds(k * 16, 16)] = rows * _C + t
    for j in range(_CHUNKS):
        pltpu.async_copy(flat_hbm.at[idx_v.at[j]], g_v.at[j], sem).wait()
    acc = jnp.zeros((16,), jnp.float32)
    for j in range(_CHUNKS):
        for k in range(8):
            acc = acc + g_v[j, pl.ds(k * 16, 16)]
    acc_v[...] = acc
    pltpu.sync_copy(acc_v, out_hbm.at[wid])


_BR = 512
_NB = _B // _BR


def _tc_body(x_ref, o_ref):
    i = pl.program_id(0)
    x = x_ref[...]
    m = jnp.max(x, axis=-1, keepdims=True)
    se = jnp.sum(jnp.exp(x - m), axis=-1)
    lse = m[:, 0] + jnp.log(se)
    sx = jnp.sum(x, axis=-1)
    part = jnp.sum(lse - _BCOEF * sx)

    @pl.when(i == 0)
    def _():
        o_ref[0, 0] = 0.0

    o_ref[0, 0] += part


def _tc_reduce(logits):
    return pl.pallas_call(
        _tc_body,
        grid=(_NB,),
        in_specs=[pl.BlockSpec((_BR, _C), lambda i: (i, 0))],
        out_specs=pl.BlockSpec((1, 1), lambda i: (0, 0)),
        out_shape=jax.ShapeDtypeStruct((1, 1), jnp.float32),
    )(logits)


def kernel(logits, targets):
    targets = targets.astype(jnp.int32)
    sc_part = _sc_gather(logits.reshape(-1), targets)
    dense = _tc_reduce(logits)
    return (dense[0, 0] - _ACOEF * jnp.sum(sc_part)) * (1.0 / _B)


# speedup vs baseline: 1.3277x; 1.3277x over previous
import functools

import jax
import jax.numpy as jnp
from jax import lax
from jax.experimental import pallas as pl
from jax.experimental.pallas import tpu as pltpu
from jax.experimental.pallas import tpu_sc as plsc

_B = 16384
_C = 1000
_EPS = 0.1
_BCOEF = _EPS / (_C - 1)
_ACOEF = (1.0 - _EPS) - _BCOEF

_NC = 2
_NS = 16
_NW = _NC * _NS
_RPW = _B // _NW
_CHUNKS = _RPW // 128


@functools.partial(
    pl.kernel,
    mesh=plsc.VectorSubcoreMesh(core_axis_name="c", subcore_axis_name="s"),
    out_type=jax.ShapeDtypeStruct((_NW, 16), jnp.float32),
    scratch_types=[
        pltpu.VMEM((_RPW,), jnp.int32),
        pltpu.VMEM((_CHUNKS, 128), jnp.int32),
        pltpu.VMEM((_CHUNKS, 128), jnp.float32),
        pltpu.VMEM((16,), jnp.float32),
        pltpu.SemaphoreType.DMA,
    ],
)
def _sc_gather(flat_hbm, tgt_hbm, out_hbm, tgt_v, idx_v, g_v, acc_v, sem):
    wid = lax.axis_index("s") * _NC + lax.axis_index("c")
    base = wid * _RPW
    pltpu.sync_copy(tgt_hbm.at[pl.ds(base, _RPW)], tgt_v)
    for j in range(_CHUNKS):
        for k in range(8):
            i = j * 8 + k
            t = tgt_v[pl.ds(i * 16, 16)]
            rows = base + i * 16 + lax.iota(jnp.int32, 16)
            idx_v[j, pl.ds(k * 16, 16)] = rows * _C + t
    for j in range(_CHUNKS):
        pltpu.async_copy(flat_hbm.at[idx_v.at[j]], g_v.at[j], sem).wait()
    acc = jnp.zeros((16,), jnp.float32)
    for j in range(_CHUNKS):
        for k in range(8):
            acc = acc + g_v[j, pl.ds(k * 16, 16)]
    acc_v[...] = acc
    pltpu.sync_copy(acc_v, out_hbm.at[wid])


_BR = 512
_NB = _B // _BR


def _tc_body(x_ref, o_ref):
    i = pl.program_id(0)
    x = x_ref[...]
    m = jnp.max(x, axis=-1, keepdims=True)
    se = jnp.sum(jnp.exp(x - m), axis=-1)
    lse = m[:, 0] + jnp.log(se)
    sx = jnp.sum(x, axis=-1)
    part = jnp.sum(lse - _BCOEF * sx)

    @pl.when(i == 0)
    def _():
        o_ref[...] = jnp.zeros((1, 1), jnp.float32)

    o_ref[...] = o_ref[...] + part


def _tc_reduce(logits):
    return pl.pallas_call(
        _tc_body,
        grid=(_NB,),
        in_specs=[pl.BlockSpec((_BR, _C), lambda i: (i, 0))],
        out_specs=pl.BlockSpec((1, 1), lambda i: (0, 0)),
        out_shape=jax.ShapeDtypeStruct((1, 1), jnp.float32),
    )(logits)


def kernel(logits, targets):
    targets = targets.astype(jnp.int32)
    sc_part = _sc_gather(logits.reshape(-1), targets)
    dense = _tc_reduce(logits)
    return (dense[0, 0] - _ACOEF * jnp.sum(sc_part)) * (1.0 / _B)


# speedup vs baseline: 1.4320x; 1.0785x over previous
import functools

import jax
import jax.numpy as jnp
from jax import lax
from jax.experimental import pallas as pl
from jax.experimental.pallas import tpu as pltpu
from jax.experimental.pallas import tpu_sc as plsc

_B = 16384
_C = 1000
_EPS = 0.1
_BCOEF = _EPS / (_C - 1)
_ACOEF = (1.0 - _EPS) - _BCOEF

_NC = 2
_NS = 16
_NW = _NC * _NS
_RPW = _B // _NW
_CHUNKS = _RPW // 128


@functools.partial(
    pl.kernel,
    mesh=plsc.VectorSubcoreMesh(core_axis_name="c", subcore_axis_name="s"),
    out_type=jax.ShapeDtypeStruct((_NW, 16), jnp.float32),
    scratch_types=[
        pltpu.VMEM((_RPW,), jnp.int32),
        pltpu.VMEM((_CHUNKS, 128), jnp.int32),
        pltpu.VMEM((_CHUNKS, 128), jnp.float32),
        pltpu.VMEM((16,), jnp.float32),
        pltpu.SemaphoreType.DMA,
    ],
)
def _sc_gather(flat_hbm, tgt_hbm, out_hbm, tgt_v, idx_v, g_v, acc_v, sem):
    wid = lax.axis_index("s") * _NC + lax.axis_index("c")
    base = wid * _RPW
    pltpu.sync_copy(tgt_hbm.at[pl.ds(base, _RPW)], tgt_v)
    for j in range(_CHUNKS):
        for k in range(8):
            i = j * 8 + k
            t = tgt_v[pl.ds(i * 16, 16)]
            rows = base + i * 16 + lax.iota(jnp.int32, 16)
            idx_v[j, pl.ds(k * 16, 16)] = rows * _C + t
    for j in range(_CHUNKS):
        pltpu.async_copy(flat_hbm.at[idx_v.at[j]], g_v.at[j], sem).wait()
    acc = jnp.zeros((16,), jnp.float32)
    for j in range(_CHUNKS):
        for k in range(8):
            acc = acc + g_v[j, pl.ds(k * 16, 16)]
    acc_v[...] = acc
    pltpu.sync_copy(acc_v, out_hbm.at[wid])


_BR = 2048
_NB = _B // _BR


def _tc_body(x_ref, o_ref):
    i = pl.program_id(0)
    x = x_ref[...]
    m = jnp.max(x, axis=-1, keepdims=True)
    se = jnp.sum(jnp.exp(x - m), axis=-1)
    lse = m[:, 0] + jnp.log(se)
    sx = jnp.sum(x, axis=-1)
    part = jnp.sum(lse - _BCOEF * sx)

    @pl.when(i == 0)
    def _():
        o_ref[...] = jnp.zeros((1, 1), jnp.float32)

    o_ref[...] = o_ref[...] + part


def _tc_reduce(logits):
    return pl.pallas_call(
        _tc_body,
        grid=(_NB,),
        in_specs=[pl.BlockSpec((_BR, _C), lambda i: (i, 0))],
        out_specs=pl.BlockSpec((1, 1), lambda i: (0, 0)),
        out_shape=jax.ShapeDtypeStruct((1, 1), jnp.float32),
    )(logits)


def kernel(logits, targets):
    targets = targets.astype(jnp.int32)
    sc_part = _sc_gather(logits.reshape(-1), targets)
    dense = _tc_reduce(logits)
    return (dense[0, 0] - _ACOEF * jnp.sum(sc_part)) * (1.0 / _B)


# speedup vs baseline: 1.4331x; 1.0008x over previous
import functools

import jax
import jax.numpy as jnp
from jax import lax
from jax.experimental import pallas as pl
from jax.experimental.pallas import tpu as pltpu
from jax.experimental.pallas import tpu_sc as plsc

_B = 16384
_C = 1000
_EPS = 0.1
_BCOEF = _EPS / (_C - 1)
_ACOEF = (1.0 - _EPS) - _BCOEF

_NC = 2
_NS = 16
_NW = _NC * _NS
_RPW = _B // _NW
_CHUNKS = _RPW // 128


@functools.partial(
    pl.kernel,
    mesh=plsc.VectorSubcoreMesh(core_axis_name="c", subcore_axis_name="s"),
    out_type=jax.ShapeDtypeStruct((_NW, 16), jnp.float32),
    scratch_types=[
        pltpu.VMEM((_RPW,), jnp.int32),
        pltpu.VMEM((_CHUNKS, 128), jnp.int32),
        pltpu.VMEM((_CHUNKS, 128), jnp.float32),
        pltpu.VMEM((16,), jnp.float32),
        pltpu.SemaphoreType.DMA,
    ],
)
def _sc_gather(flat_hbm, tgt_hbm, out_hbm, tgt_v, idx_v, g_v, acc_v, sem):
    wid = lax.axis_index("s") * _NC + lax.axis_index("c")
    base = wid * _RPW
    pltpu.sync_copy(tgt_hbm.at[pl.ds(base, _RPW)], tgt_v)
    for j in range(_CHUNKS):
        for k in range(8):
            i = j * 8 + k
            t = tgt_v[pl.ds(i * 16, 16)]
            rows = base + i * 16 + lax.iota(jnp.int32, 16)
            idx_v[j, pl.ds(k * 16, 16)] = rows * _C + t
    for j in range(_CHUNKS):
        pltpu.async_copy(flat_hbm.at[idx_v.at[j]], g_v.at[j], sem).wait()
    acc = jnp.zeros((16,), jnp.float32)
    for j in range(_CHUNKS):
        for k in range(8):
            acc = acc + g_v[j, pl.ds(k * 16, 16)]
    acc_v[...] = acc
    pltpu.sync_copy(acc_v, out_hbm.at[wid])


_BR = 512
_NSTREAM = 4
_NB = _B // (_BR * _NSTREAM)


def _block_part(x):
    m = jnp.max(x, axis=-1, keepdims=True)
    se = jnp.sum(jnp.exp(x - m), axis=-1)
    lse = m[:, 0] + jnp.log(se)
    sx = jnp.sum(x, axis=-1)
    return jnp.sum(lse - _BCOEF * sx)


def _tc_body(*refs):
    o_ref = refs[-1]
    i = pl.program_id(0)
    part = _block_part(refs[0][...])
    for k in range(1, _NSTREAM):
        part += _block_part(refs[k][...])

    @pl.when(i == 0)
    def _():
        o_ref[...] = jnp.zeros((1, 1), jnp.float32)

    o_ref[...] = o_ref[...] + part


def _tc_reduce(logits):
    return pl.pallas_call(
        _tc_body,
        grid=(_NB,),
        in_specs=[
            pl.BlockSpec((_BR, _C), functools.partial(lambda k, i: (_NSTREAM * i + k, 0), k))
            for k in range(_NSTREAM)
        ],
        out_specs=pl.BlockSpec((1, 1), lambda i: (0, 0)),
        out_shape=jax.ShapeDtypeStruct((1, 1), jnp.float32),
    )(*([logits] * _NSTREAM))


def kernel(logits, targets):
    targets = targets.astype(jnp.int32)
    sc_part = _sc_gather(logits.reshape(-1), targets)
    dense = _tc_reduce(logits)
    return (dense[0, 0] - _ACOEF * jnp.sum(sc_part)) * (1.0 / _B)


# speedup vs baseline: 2.8832x; 2.0118x over previous
import functools

import jax
import jax.numpy as jnp
from jax import lax
from jax.experimental import pallas as pl
from jax.experimental.pallas import tpu as pltpu
from jax.experimental.pallas import tpu_sc as plsc

_B = 16384
_C = 1000
_EPS = 0.1
_BCOEF = _EPS / (_C - 1)
_ACOEF = (1.0 - _EPS) - _BCOEF

_NC = 2
_NS = 16
_NW = _NC * _NS
_RPW = _B // _NW
_CHUNKS = _RPW // 128


@functools.partial(
    pl.kernel,
    mesh=plsc.VectorSubcoreMesh(core_axis_name="c", subcore_axis_name="s"),
    out_type=jax.ShapeDtypeStruct((_NW, 16), jnp.float32),
    scratch_types=[
        pltpu.VMEM((_RPW,), jnp.int32),
        pltpu.VMEM((_CHUNKS, 128), jnp.int32),
        pltpu.VMEM((_CHUNKS, 128), jnp.float32),
        pltpu.VMEM((16,), jnp.float32),
        pltpu.SemaphoreType.DMA,
    ],
)
def _sc_gather(flat_hbm, tgt_hbm, out_hbm, tgt_v, idx_v, g_v, acc_v, sem):
    wid = lax.axis_index("s") * _NC + lax.axis_index("c")
    base = wid * _RPW
    pltpu.sync_copy(tgt_hbm.at[pl.ds(base, _RPW)], tgt_v)
    for j in range(_CHUNKS):
        for k in range(8):
            i = j * 8 + k
            t = tgt_v[pl.ds(i * 16, 16)]
            rows = base + i * 16 + lax.iota(jnp.int32, 16)
            idx_v[j, pl.ds(k * 16, 16)] = rows * _C + t
    for j in range(_CHUNKS):
        pltpu.async_copy(flat_hbm.at[idx_v.at[j]], g_v.at[j], sem).wait()
    acc = jnp.zeros((16,), jnp.float32)
    for j in range(_CHUNKS):
        for k in range(8):
            acc = acc + g_v[j, pl.ds(k * 16, 16)]
    acc_v[...] = acc
    pltpu.sync_copy(acc_v, out_hbm.at[wid])


_BR = 512
_NSTREAM = 4
_NB = _B // (_BR * _NSTREAM)


def _block_part(x, t):
    m = jnp.max(x, axis=-1, keepdims=True)
    se = jnp.sum(jnp.exp(x - m), axis=-1)
    lse = m[:, 0] + jnp.log(se)
    sx = jnp.sum(x, axis=-1)
    cols = lax.broadcasted_iota(jnp.int32, (_BR, _C), 1)
    g = jnp.sum(jnp.where(cols == t[0, 0, :, None], x, 0.0), axis=-1)
    return jnp.sum(lse - _BCOEF * sx - _ACOEF * g)


def _tc_body(*refs):
    o_ref = refs[-1]
    i = pl.program_id(0)
    part = _block_part(refs[0][...], refs[_NSTREAM][...])
    for k in range(1, _NSTREAM):
        part += _block_part(refs[k][...], refs[_NSTREAM + k][...])

    @pl.when(i == 0)
    def _():
        o_ref[...] = jnp.zeros((1, 1), jnp.float32)

    o_ref[...] = o_ref[...] + part


def _tc_reduce(logits, targets3):
    return pl.pallas_call(
        _tc_body,
        grid=(_NB,),
        in_specs=[
            pl.BlockSpec((_BR, _C), functools.partial(lambda k, i: (_NSTREAM * i + k, 0), k))
            for k in range(_NSTREAM)
        ] + [
            pl.BlockSpec((1, 1, _BR), functools.partial(lambda k, i: (_NSTREAM * i + k, 0, 0), k))
            for k in range(_NSTREAM)
        ],
        out_specs=pl.BlockSpec((1, 1), lambda i: (0, 0)),
        out_shape=jax.ShapeDtypeStruct((1, 1), jnp.float32),
    )(*([logits] * _NSTREAM + [targets3] * _NSTREAM))


def kernel(logits, targets):
    targets3 = targets.astype(jnp.int32).reshape(_B // _BR, 1, _BR)
    dense = _tc_reduce(logits, targets3)
    return dense[0, 0] * (1.0 / _B)


# speedup vs baseline: 7.9746x; 2.7659x over previous
import functools

import jax
import jax.numpy as jnp
from jax import lax
from jax.experimental import pallas as pl
from jax.experimental.pallas import tpu as pltpu

_B = 16384
_C = 1000
_EPS = 0.1
_BCOEF = _EPS / (_C - 1)
_ACOEF = (1.0 - _EPS) - _BCOEF

_BCOL = 2048
_NB = _B // _BCOL


def _tc_body(x_ref, t_ref, o_ref):
    i = pl.program_id(0)
    x = x_ref[...]
    tt = t_ref[0, 0, :]
    m = jnp.max(x, axis=0)
    se = jnp.sum(jnp.exp(x - m[None, :]), axis=0)
    lse = m + jnp.log(se)
    sx = jnp.sum(x, axis=0)
    rows = lax.broadcasted_iota(jnp.int32, (_C, _BCOL), 0)
    g = jnp.sum(jnp.where(rows == tt[None, :], x, 0.0), axis=0)
    part = jnp.sum(lse - _BCOEF * sx - _ACOEF * g)

    @pl.when(i == 0)
    def _():
        o_ref[...] = jnp.zeros((1, 1), jnp.float32)

    o_ref[...] = o_ref[...] + part


def _tc_reduce(logits_t, targets3):
    return pl.pallas_call(
        _tc_body,
        grid=(_NB,),
        in_specs=[
            pl.BlockSpec((_C, _BCOL), lambda i: (0, i)),
            pl.BlockSpec((1, 1, _BCOL), lambda i: (i, 0, 0)),
        ],
        out_specs=pl.BlockSpec((1, 1), lambda i: (0, 0)),
        out_shape=jax.ShapeDtypeStruct((1, 1), jnp.float32),
    )(logits_t, targets3)


def kernel(logits, targets):
    targets3 = targets.astype(jnp.int32).reshape(_NB, 1, _BCOL)
    dense = _tc_reduce(logits.T, targets3)
    return dense[0, 0] * (1.0 / _B)
